# baseline (speedup 1.0000x reference)
import jax
import jax.numpy as jnp
from jax import lax
from jax.experimental import pallas as pl
from jax.experimental.pallas import tpu as pltpu
from jax.experimental.pallas import tpu_sc as plsc

N = 100000
D = 128
NW = 32
LANES = 16
RB = 3120
EXTRA = 10
RMAX = RB + 16
CHUNK = 240
NCHUNK = RB // CHUNK
NGROUP = CHUNK // LANES
NBUF = 3
NEG = -1.0e30
PAD = -3.0e38

_mesh = plsc.VectorSubcoreMesh(core_axis_name="c", subcore_axis_name="s")


def _wid():
    return lax.axis_index("s") * 2 + lax.axis_index("c")


def _k1_body(emb, w_hbm, b_hbm, sil_hbm, dec_hbm,
             scores_out, lmax_out, lsum_out,
             buf, w_v, b_v, sil_v, dec_v, scores_v, stat_v,
             semring, sem_s, sem_d):
    wid = _wid()
    start = wid * RB + 16 * jnp.minimum(wid, EXTRA)
    CD = CHUNK * D

    def issue(cin):
        sel = cin % NBUF
        pltpu.async_copy(emb.at[pl.ds((start + cin * CHUNK) * D, CD)],
                         buf.at[pl.ds(sel * CD, CD)], semring.at[sel])

    def wait_for(cin):
        sel = cin % NBUF
        pltpu.make_async_copy(emb.at[pl.ds(0, CD)],
                              buf.at[pl.ds(sel * CD, CD)],
                              semring.at[sel]).wait()

    def prebody(c0, carry):
        issue(c0)
        return carry

    lax.fori_loop(0, NBUF - 1, prebody, 0)
    pltpu.sync_copy(w_hbm, w_v)
    pltpu.sync_copy(b_hbm, b_v)
    cp_s = pltpu.async_copy(sil_hbm.at[pl.ds(start, RB)],
                            sil_v.at[pl.ds(0, RB)], sem_s)
    cp_d = pltpu.async_copy(dec_hbm.at[pl.ds(start, RB)],
                            dec_v.at[pl.ds(0, RB)], sem_d)

    wvs = [w_v[pl.ds(k * LANES, LANES)] for k in range(D // LANES)]
    b_vec = b_v[...]
    lane = lax.iota(jnp.int32, LANES)
    cp_s.wait()
    cp_d.wait()

    def masked_group(res, off):
        res = res + b_vec
        res = res + jnp.where(sil_v[pl.ds(off, LANES)] == 1, NEG, 0.0)
        res = res + jnp.where(dec_v[pl.ds(off, LANES)] == 1, NEG, 0.0)
        return res

    half = lane < 8

    def score_group(rowbase):
        res = jnp.zeros((LANES,), jnp.float32)
        for l2 in range(LANES // 2):
            ps = []
            for l in (2 * l2, 2 * l2 + 1):
                rb = rowbase + l * D
                p = buf[pl.ds(rb, LANES)] * wvs[0]
                for k in range(1, D // LANES):
                    p = p + buf[pl.ds(rb + k * LANES, LANES)] * wvs[k]
                ps.append(p + lax.rev(p, (0,)))
            c = plsc.cumsum(jnp.where(half, ps[0], ps[1]))
            sa = c[7]
            sb = c[15] - c[7]
            res = jnp.where(lane == 2 * l2, sa, res)
            res = jnp.where(lane == 2 * l2 + 1, sb, res)
        return res

    def chunk_body(ci, runmax):
        @pl.when(ci + (NBUF - 1) < NCHUNK)
        def _():
            issue(ci + (NBUF - 1))

        wait_for(ci)
        bufbase = (ci % NBUF) * CD

        def gbody(g, mx):
            off = ci * CHUNK + g * LANES
            res = masked_group(score_group(bufbase + g * LANES * D), off)
            scores_v[pl.ds(off, LANES)] = res
            return jnp.maximum(mx, res)

        return lax.fori_loop(0, NGROUP, gbody, runmax)

    runmax = lax.fori_loop(0, NCHUNK, chunk_body,
                           jnp.full((LANES,), PAD, jnp.float32))

    @pl.when(wid < EXTRA)
    def _():
        pltpu.sync_copy(emb.at[pl.ds((start + RB) * D, LANES * D)],
                        buf.at[pl.ds(0, LANES * D)])
        pltpu.sync_copy(sil_hbm.at[pl.ds(start + RB, LANES)],
                        sil_v.at[pl.ds(RB, LANES)])
        pltpu.sync_copy(dec_hbm.at[pl.ds(start + RB, LANES)],
                        dec_v.at[pl.ds(RB, LANES)])
        scores_v[pl.ds(RB, LANES)] = masked_group(score_group(0), RB)

    @pl.when(wid >= EXTRA)
    def _():
        scores_v[pl.ds(RB, LANES)] = jnp.full((LANES,), PAD, jnp.float32)

    runmax = jnp.maximum(runmax, scores_v[pl.ds(RB, LANES)])
    lmax = jnp.max(runmax)

    def ebody(k, ac):
        return ac + jnp.exp(scores_v[pl.ds(k * LANES, LANES)] - lmax)

    es = lax.fori_loop(0, RMAX // LANES, ebody,
                       jnp.zeros((LANES,), jnp.float32))
    lsum = jnp.sum(es)

    stat_v[...] = jnp.full((LANES,), lmax, jnp.float32)
    pltpu.sync_copy(stat_v, lmax_out.at[wid])
    stat_v[...] = jnp.full((LANES,), lsum, jnp.float32)
    pltpu.sync_copy(stat_v, lsum_out.at[wid])

    pltpu.sync_copy(scores_v.at[pl.ds(0, RB)], scores_out.at[pl.ds(start, RB)])

    @pl.when(wid < EXTRA)
    def _():
        pltpu.sync_copy(scores_v.at[pl.ds(RB, LANES)],
                        scores_out.at[pl.ds(start + RB, LANES)])


_k1 = pl.kernel(
    _k1_body,
    out_type=[
        jax.ShapeDtypeStruct((N,), jnp.float32),
        jax.ShapeDtypeStruct((NW, LANES), jnp.float32),
        jax.ShapeDtypeStruct((NW, LANES), jnp.float32),
    ],
    mesh=_mesh,
    compiler_params=pltpu.CompilerParams(needs_layout_passes=False),
    scratch_types=[
        pltpu.VMEM((NBUF * CHUNK * D,), jnp.float32),
        pltpu.VMEM((D,), jnp.float32),
        pltpu.VMEM((LANES,), jnp.float32),
        pltpu.VMEM((RMAX,), jnp.int32),
        pltpu.VMEM((RMAX,), jnp.int32),
        pltpu.VMEM((RMAX,), jnp.float32),
        pltpu.VMEM((LANES,), jnp.float32),
        pltpu.SemaphoreType.DMA((NBUF,)),
        pltpu.SemaphoreType.DMA,
        pltpu.SemaphoreType.DMA,
    ],
)

FBLK = 512
FGRID = -(-N // FBLK)


def _fin_body(lmax_ref, lsum_ref, sc_ref, out_ref, gstat):
    i = pl.program_id(0)

    @pl.when(i == 0)
    def _():
        gmax = jnp.max(lmax_ref[...])
        e = lsum_ref[:, 0:1] * jnp.exp(lmax_ref[:, 0:1] - gmax)
        gstat[0] = gmax
        gstat[1] = 1.0 / jnp.sum(e)

    out_ref[...] = jnp.exp(sc_ref[...] - gstat[0]) * gstat[1]


_fin = pl.pallas_call(
    _fin_body,
    grid=(FGRID,),
    in_specs=[
        pl.BlockSpec((NW, LANES), lambda i: (0, 0)),
        pl.BlockSpec((NW, LANES), lambda i: (0, 0)),
        pl.BlockSpec((FBLK, 1), lambda i: (i, 0)),
    ],
    out_specs=pl.BlockSpec((FBLK, 1), lambda i: (i, 0)),
    out_shape=jax.ShapeDtypeStruct((N, 1), jnp.float32),
    scratch_shapes=[pltpu.SMEM((2,), jnp.float32)],
)


@jax.jit
def kernel(embeddings, W, b, silent_np, decision, number_of_candidates):
    del number_of_candidates
    w = W.reshape(D)
    b16 = jnp.broadcast_to(b.reshape(()), (LANES,))
    scores, lmaxs, lsums = _k1(embeddings.reshape(N * D), w, b16,
                               silent_np, decision)
    return _fin(lmaxs, lsums, scores.reshape(N, 1)).reshape(N)


# speedup vs baseline: 4.3085x; 4.3085x over previous
import jax
import jax.numpy as jnp
from jax import lax
from jax.experimental import pallas as pl
from jax.experimental.pallas import tpu as pltpu
from jax.experimental.pallas import tpu_sc as plsc

N = 100000
D = 128
NW = 32
LANES = 16
RB = 3120
EXTRA = 10
RMAX = RB + 16
CHUNK = 240
NCHUNK = RB // CHUNK
NGROUP = CHUNK // LANES
NBUF = 3
NEG = -1.0e30
PAD = -3.0e38

_mesh = plsc.VectorSubcoreMesh(core_axis_name="c", subcore_axis_name="s")


def _wid():
    return lax.axis_index("s") * 2 + lax.axis_index("c")


def _k1_body(emb, w_hbm, b_hbm, sil_hbm, dec_hbm,
             scores_out, lmax_out, lsum_out,
             buf, w_v, b_v, sil_v, dec_v, scores_v, stat_v,
             semring, sem_s, sem_d):
    wid = _wid()
    start = wid * RB + 16 * jnp.minimum(wid, EXTRA)
    CD = CHUNK * D

    def issue(cin):
        sel = cin % NBUF
        pltpu.async_copy(emb.at[pl.ds((start + cin * CHUNK) * D, CD)],
                         buf.at[pl.ds(sel * CD, CD)], semring.at[sel])

    def wait_for(cin):
        sel = cin % NBUF
        pltpu.make_async_copy(emb.at[pl.ds(0, CD)],
                              buf.at[pl.ds(sel * CD, CD)],
                              semring.at[sel]).wait()

    def prebody(c0, carry):
        issue(c0)
        return carry

    lax.fori_loop(0, NBUF - 1, prebody, 0)
    pltpu.sync_copy(w_hbm, w_v)
    pltpu.sync_copy(b_hbm, b_v)
    cp_s = pltpu.async_copy(sil_hbm.at[pl.ds(start, RB)],
                            sil_v.at[pl.ds(0, RB)], sem_s)
    cp_d = pltpu.async_copy(dec_hbm.at[pl.ds(start, RB)],
                            dec_v.at[pl.ds(0, RB)], sem_d)

    wvs = [w_v[pl.ds(k * LANES, LANES)] for k in range(D // LANES)]
    b_vec = b_v[...]
    lane = lax.iota(jnp.int32, LANES)
    cp_s.wait()
    cp_d.wait()

    def masked_group(res, off):
        res = res + b_vec
        res = res + jnp.where(sil_v[pl.ds(off, LANES)] == 1, NEG, 0.0)
        res = res + jnp.where(dec_v[pl.ds(off, LANES)] == 1, NEG, 0.0)
        return res

    half = lane < 8

    def score_group(rowbase):
        res = jnp.zeros((LANES,), jnp.float32)
        for l2 in range(LANES // 2):
            ps = []
            for l in (2 * l2, 2 * l2 + 1):
                rb = rowbase + l * D
                p = buf[pl.ds(rb, LANES)] * wvs[0]
                for k in range(1, D // LANES):
                    p = p + buf[pl.ds(rb + k * LANES, LANES)] * wvs[k]
                ps.append(p + lax.rev(p, (0,)))
            c = plsc.cumsum(jnp.where(half, ps[0], ps[1]))
            sa = c[7]
            sb = c[15] - c[7]
            res = jnp.where(lane == 2 * l2, sa, res)
            res = jnp.where(lane == 2 * l2 + 1, sb, res)
        return res

    def chunk_body(ci, runmax):
        @pl.when(ci + (NBUF - 1) < NCHUNK)
        def _():
            issue(ci + (NBUF - 1))

        wait_for(ci)
        bufbase = (ci % NBUF) * CD

        def gbody(g, mx):
            off = ci * CHUNK + g * LANES
            res = masked_group(score_group(bufbase + g * LANES * D), off)
            scores_v[pl.ds(off, LANES)] = res
            return jnp.maximum(mx, res)

        return lax.fori_loop(0, NGROUP, gbody, runmax)

    runmax = lax.fori_loop(0, NCHUNK, chunk_body,
                           jnp.full((LANES,), PAD, jnp.float32))

    @pl.when(wid < EXTRA)
    def _():
        pltpu.sync_copy(emb.at[pl.ds((start + RB) * D, LANES * D)],
                        buf.at[pl.ds(0, LANES * D)])
        pltpu.sync_copy(sil_hbm.at[pl.ds(start + RB, LANES)],
                        sil_v.at[pl.ds(RB, LANES)])
        pltpu.sync_copy(dec_hbm.at[pl.ds(start + RB, LANES)],
                        dec_v.at[pl.ds(RB, LANES)])
        scores_v[pl.ds(RB, LANES)] = masked_group(score_group(0), RB)

    @pl.when(wid >= EXTRA)
    def _():
        scores_v[pl.ds(RB, LANES)] = jnp.full((LANES,), PAD, jnp.float32)

    runmax = jnp.maximum(runmax, scores_v[pl.ds(RB, LANES)])
    lmax = jnp.max(runmax)

    def ebody(k, ac):
        return ac + jnp.exp(scores_v[pl.ds(k * LANES, LANES)] - lmax)

    es = lax.fori_loop(0, RMAX // LANES, ebody,
                       jnp.zeros((LANES,), jnp.float32))
    lsum = jnp.sum(es)

    stat_v[...] = jnp.full((LANES,), lmax, jnp.float32)
    pltpu.sync_copy(stat_v, lmax_out.at[wid])
    stat_v[...] = jnp.full((LANES,), lsum, jnp.float32)
    pltpu.sync_copy(stat_v, lsum_out.at[wid])

    pltpu.sync_copy(scores_v.at[pl.ds(0, RB)], scores_out.at[pl.ds(start, RB)])

    @pl.when(wid < EXTRA)
    def _():
        pltpu.sync_copy(scores_v.at[pl.ds(RB, LANES)],
                        scores_out.at[pl.ds(start + RB, LANES)])


_k1 = pl.kernel(
    _k1_body,
    out_type=[
        jax.ShapeDtypeStruct((N,), jnp.float32),
        jax.ShapeDtypeStruct((NW, LANES), jnp.float32),
        jax.ShapeDtypeStruct((NW, LANES), jnp.float32),
    ],
    mesh=_mesh,
    compiler_params=pltpu.CompilerParams(needs_layout_passes=False),
    scratch_types=[
        pltpu.VMEM((NBUF * CHUNK * D,), jnp.float32),
        pltpu.VMEM((D,), jnp.float32),
        pltpu.VMEM((LANES,), jnp.float32),
        pltpu.VMEM((RMAX,), jnp.int32),
        pltpu.VMEM((RMAX,), jnp.int32),
        pltpu.VMEM((RMAX,), jnp.float32),
        pltpu.VMEM((LANES,), jnp.float32),
        pltpu.SemaphoreType.DMA((NBUF,)),
        pltpu.SemaphoreType.DMA,
        pltpu.SemaphoreType.DMA,
    ],
)

FPAD = 784 * 128


def _fin_body(lmax_ref, lsum_ref, sc_ref, out_ref):
    gmax = jnp.max(lmax_ref[...])
    e = lsum_ref[:, 0:1] * jnp.exp(lmax_ref[:, 0:1] - gmax)
    inv = 1.0 / jnp.sum(e)
    out_ref[...] = jnp.exp(sc_ref[...] - gmax) * inv


_fin = pl.pallas_call(
    _fin_body,
    out_shape=jax.ShapeDtypeStruct((FPAD // 128, 128), jnp.float32),
)


@jax.jit
def kernel(embeddings, W, b, silent_np, decision, number_of_candidates):
    del number_of_candidates
    w = W.reshape(D)
    b16 = jnp.broadcast_to(b.reshape(()), (LANES,))
    scores, lmaxs, lsums = _k1(embeddings.reshape(N * D), w, b16,
                               silent_np, decision)
    sc_pad = jnp.pad(scores, (0, FPAD - N)).reshape(FPAD // 128, 128)
    return _fin(lmaxs, lsums, sc_pad).reshape(FPAD)[:N]
